# Initial kernel scaffold; baseline (speedup 1.0000x reference)
import jax
import jax.numpy as jnp
from jax.experimental import pallas as pl


def kernel(inputs, weight):
    raise NotImplementedError("write your pallas kernel here")


# speedup vs baseline: 1.9791x; 1.9791x over previous
import jax
import jax.numpy as jnp
from jax.experimental import pallas as pl

_K = 1024
_D = 64
_N = 16 * 1024
_R = 512
_GRID = _N // _R


def _vq_body(x_ref, w_ref, dist_ref, enc_ref, q_ref, idx_ref,
             counts_ref, loss_ref):
    i = pl.program_id(0)
    xb = x_ref[...]
    w = w_ref[...]
    xsq = jnp.sum(xb * xb, axis=1, keepdims=True)
    wsq = jnp.sum(w * w, axis=1)[None, :]
    mm = jax.lax.dot_general(
        xb, w, dimension_numbers=(((1,), (1,)), ((), ())),
        preferred_element_type=jnp.float32)
    dist = xsq + wsq - 2.0 * mm
    dist_ref[...] = dist

    mind = jnp.min(dist, axis=1, keepdims=True)
    kiota = jax.lax.broadcasted_iota(jnp.int32, (_R, _K), 1)
    idx = jnp.min(jnp.where(dist == mind, kiota, _K), axis=1)
    idx_ref[...] = idx[:, None]

    one_hot = (kiota == idx[:, None]).astype(jnp.float32)
    enc_ref[...] = one_hot

    q = jnp.dot(one_hot, w, preferred_element_type=jnp.float32)
    q_ref[...] = q

    @pl.when(i == 0)
    def _init():
        counts_ref[...] = jnp.zeros_like(counts_ref)
        loss_ref[...] = jnp.zeros_like(loss_ref)

    counts_ref[...] += jnp.sum(one_hot, axis=0, keepdims=True)
    s = jnp.sum((q - xb) ** 2)
    lane = jax.lax.broadcasted_iota(jnp.int32, (1, 128), 1)
    loss_ref[...] += jnp.where(lane == 0, s, 0.0)


def kernel(inputs, weight):
    x = jnp.transpose(inputs, (0, 2, 1)).reshape(_N, _D)

    dist, enc, q, idx, counts, losspart = pl.pallas_call(
        _vq_body,
        grid=(_GRID,),
        in_specs=[
            pl.BlockSpec((_R, _D), lambda i: (i, 0)),
            pl.BlockSpec((_K, _D), lambda i: (0, 0)),
        ],
        out_specs=[
            pl.BlockSpec((_R, _K), lambda i: (i, 0)),
            pl.BlockSpec((_R, _K), lambda i: (i, 0)),
            pl.BlockSpec((_R, _D), lambda i: (i, 0)),
            pl.BlockSpec((_R, 1), lambda i: (i, 0)),
            pl.BlockSpec((1, _K), lambda i: (0, 0)),
            pl.BlockSpec((1, 128), lambda i: (0, 0)),
        ],
        out_shape=[
            jax.ShapeDtypeStruct((_N, _K), jnp.float32),
            jax.ShapeDtypeStruct((_N, _K), jnp.float32),
            jax.ShapeDtypeStruct((_N, _D), jnp.float32),
            jax.ShapeDtypeStruct((_N, 1), jnp.int32),
            jax.ShapeDtypeStruct((1, _K), jnp.float32),
            jax.ShapeDtypeStruct((1, 128), jnp.float32),
        ],
    )(x, weight)

    n_elems = jnp.float32(_N * _D)
    e_latent = losspart[0, 0] / n_elems
    vq_loss = e_latent + 0.25 * e_latent

    avg_probs = counts[0] / jnp.float32(_N)
    perplexity = jnp.exp(-jnp.sum(avg_probs * jnp.log(avg_probs + 1e-10)))

    quantized_st = jnp.transpose(q.reshape(16, 1024, _D), (0, 2, 1))
    encodings = enc.reshape(_D, 1024, -1)
    distances = dist.reshape(_D, 1024, -1)
    return (vq_loss, quantized_st, perplexity, encodings, distances, idx)
